# baseline (speedup 1.0000x reference)
import numpy as np
import jax
import jax.numpy as jnp
from jax import lax
from jax.experimental import pallas as pl
from jax.experimental.pallas import tpu as pltpu
from jax.experimental.pallas import tpu_sc as plsc

_T = 1000
_TPAD = 1024


def _make_tables():
    beta = np.linspace(0.0001, 0.02, _T, dtype=np.float32)
    alpha = (1.0 - beta).astype(np.float32)
    alpha_hat = np.cumprod(alpha, dtype=np.float32)
    sa = np.zeros(_TPAD, np.float32)
    sb = np.zeros(_TPAD, np.float32)
    sa[:_T] = np.sqrt(alpha_hat)
    sb[:_T] = np.sqrt((1.0 - alpha_hat).astype(np.float32))
    return sa, sb


_SA, _SB = _make_tables()

_B = 256
_N = 3 * 128 * 128
_NW = 32
_RPW = _B // _NW
_CPR = 4
_CH = _N // _CPR
_NCH = _RPW * _CPR
_L = 16


def _sc_body(ts_hbm, sa_hbm, sb_hbm, x_hbm, e_hbm, out_hbm, out2_hbm,
             idx_v, tsr_v, a_all, b_all,
             xv0, xv1, xv2, ev0, ev1, ev2, ev3, ov0, ov1, ov2,
             gsem, sinx, sine, sout, sout2):
    xv = [xv0, xv1, xv2]
    ev = [ev0, ev1, ev2, ev3]
    ov = [ov0, ov1, ov2]
    cid = lax.axis_index("c")
    sid = lax.axis_index("s")
    wid = sid * 2 + cid
    base = wid * _RPW * _N

    def issue_in(k):
        off = base + k * _CH
        pltpu.async_copy(x_hbm.at[pl.ds(off, _CH)], xv[k % 3], sinx.at[k % 3])
        pltpu.async_copy(e_hbm.at[pl.ds(off, _CH)], ev[k % 4], sine.at[k % 4])

    def wait_in(j):
        off = base + j * _CH
        pltpu.make_async_copy(x_hbm.at[pl.ds(off, _CH)], xv[j % 3], sinx.at[j % 3]).wait()
        pltpu.make_async_copy(e_hbm.at[pl.ds(off, _CH)], ev[j % 4], sine.at[j % 4]).wait()

    def issue_out(j):
        off = base + j * _CH
        pltpu.async_copy(ov[j % 3], out_hbm.at[pl.ds(off, _CH)], sout.at[j % 3])

    def wait_out(j):
        off = base + j * _CH
        pltpu.make_async_copy(ov[j % 3], out_hbm.at[pl.ds(off, _CH)], sout.at[j % 3]).wait()

    def issue_out2(j):
        off = base + j * _CH
        pltpu.async_copy(ev[j % 4], out2_hbm.at[pl.ds(off, _CH)], sout2.at[j % 4])

    def wait_out2(j):
        off = base + j * _CH
        pltpu.make_async_copy(ev[j % 4], out2_hbm.at[pl.ds(off, _CH)], sout2.at[j % 4]).wait()

    issue_in(0)
    issue_in(1)

    for r in range(_RPW):
        idx_v[pl.ds(r * _L, _L)] = jnp.full((_L,), 0, jnp.int32) + (wid * _RPW + r)
    pltpu.async_copy(ts_hbm.at[idx_v], tsr_v, gsem).wait()
    pltpu.async_copy(sa_hbm.at[tsr_v], a_all, gsem).wait()
    pltpu.async_copy(sb_hbm.at[tsr_v], b_all, gsem).wait()

    for j in range(_NCH):
        k = j + 2
        if k < _NCH:
            if k >= 4:
                wait_out2(k - 4)
            issue_in(k)
        wait_in(j)
        if j >= 3:
            wait_out(j - 3)
        xb = xv[j % 3]
        eb = ev[j % 4]
        ob = ov[j % 3]
        av = a_all[pl.ds((j // _CPR) * _L, _L)]
        bv = b_all[pl.ds((j // _CPR) * _L, _L)]

        @plsc.parallel_loop(0, _CH, _L, unroll=8)
        def _(i, xb=xb, eb=eb, ob=ob, av=av, bv=bv):
            ob[pl.ds(i, _L)] = av * xb[pl.ds(i, _L)] + bv * eb[pl.ds(i, _L)]

        issue_out(j)
        issue_out2(j)
    wait_out(_NCH - 3)
    wait_out(_NCH - 2)
    wait_out(_NCH - 1)
    wait_out2(_NCH - 4)
    wait_out2(_NCH - 3)
    wait_out2(_NCH - 2)
    wait_out2(_NCH - 1)


def kernel(x0, timesteps, eps):
    xf = x0.reshape(_B * _N)
    ef = eps.reshape(_B * _N)
    ts = timesteps.astype(jnp.int32)
    sa = jnp.asarray(_SA)
    sb = jnp.asarray(_SB)

    mesh = plsc.VectorSubcoreMesh(core_axis_name="c", subcore_axis_name="s")
    run = pl.kernel(
        _sc_body,
        mesh=mesh,
        out_type=(
            jax.ShapeDtypeStruct((_B * _N,), jnp.float32),
            jax.ShapeDtypeStruct((_B * _N,), jnp.float32),
        ),
        scratch_types=[
            pltpu.VMEM((_RPW * _L,), jnp.int32),
            pltpu.VMEM((_RPW * _L,), jnp.int32),
            pltpu.VMEM((_RPW * _L,), jnp.float32),
            pltpu.VMEM((_RPW * _L,), jnp.float32),
            pltpu.VMEM((_CH,), jnp.float32),
            pltpu.VMEM((_CH,), jnp.float32),
            pltpu.VMEM((_CH,), jnp.float32),
            pltpu.VMEM((_CH,), jnp.float32),
            pltpu.VMEM((_CH,), jnp.float32),
            pltpu.VMEM((_CH,), jnp.float32),
            pltpu.VMEM((_CH,), jnp.float32),
            pltpu.VMEM((_CH,), jnp.float32),
            pltpu.VMEM((_CH,), jnp.float32),
            pltpu.VMEM((_CH,), jnp.float32),
            pltpu.SemaphoreType.DMA,
            pltpu.SemaphoreType.DMA((3,)),
            pltpu.SemaphoreType.DMA((4,)),
            pltpu.SemaphoreType.DMA((3,)),
            pltpu.SemaphoreType.DMA((4,)),
        ],
    )
    out, out2 = run(ts, sa, sb, xf, ef)
    return (out.reshape(x0.shape), out2.reshape(x0.shape))


# speedup vs baseline: 1.6784x; 1.6784x over previous
import numpy as np
import jax
import jax.numpy as jnp
from jax.experimental import pallas as pl
from jax.experimental.pallas import tpu as pltpu

_T = 1000


def _make_tables():
    beta = np.linspace(0.0001, 0.02, _T, dtype=np.float32)
    alpha = (1.0 - beta).astype(np.float32)
    alpha_hat = np.cumprod(alpha, dtype=np.float32)
    sa = np.sqrt(alpha_hat).astype(np.float32)
    sb = np.sqrt((1.0 - alpha_hat).astype(np.float32)).astype(np.float32)
    return sa, sb


_SA, _SB = _make_tables()

_BM = 16


def _body(ts_ref, sa_ref, sb_ref, x_ref, e_ref, o_ref, o2_ref):
    base = pl.program_id(0) * _BM
    ca, cb = [], []
    for r in range(_BM):
        t = ts_ref[base + r]
        ca.append(sa_ref[t])
        cb.append(sb_ref[t])
    a = jnp.stack(ca).reshape(_BM, 1, 1, 1)
    b = jnp.stack(cb).reshape(_BM, 1, 1, 1)
    e = e_ref[...]
    o_ref[...] = a * x_ref[...] + b * e
    o2_ref[...] = e


def kernel(x0, timesteps, eps):
    B, C, H, W = x0.shape
    ts = timesteps.astype(jnp.int32)
    sa = jnp.asarray(_SA)
    sb = jnp.asarray(_SB)

    grid = (B // _BM,)
    smem = pl.BlockSpec(memory_space=pltpu.SMEM)
    blk = pl.BlockSpec((_BM, C, H, W), lambda i: (i, 0, 0, 0))
    out, out2 = pl.pallas_call(
        _body,
        grid=grid,
        in_specs=[smem, smem, smem, blk, blk],
        out_specs=(blk, blk),
        out_shape=(
            jax.ShapeDtypeStruct((B, C, H, W), jnp.float32),
            jax.ShapeDtypeStruct((B, C, H, W), jnp.float32),
        ),
    )(ts, sa, sb, x0, eps)
    return (out, out2)
